# baseline (speedup 1.0000x reference)
import jax
import jax.numpy as jnp
from jax.experimental import pallas as pl
from jax.experimental.pallas import tpu as pltpu

_S = 64


def _mm_kernel(x_ref, w_ref, o_ref):
    k = pl.program_id(1)
    xb = x_ref[...].astype(jnp.bfloat16)
    part = jax.lax.dot_general(
        xb, w_ref[...], (((1,), (0,)), ((), ())),
        preferred_element_type=jnp.float32)

    @pl.when(k == 0)
    def _init():
        o_ref[...] = part

    @pl.when(k > 0)
    def _acc():
        o_ref[...] += part


def kernel(x, start_boundaries, W_loc, b_loc, W_conf, b_conf):
    B, C = x.shape[0], x.shape[1]
    K = C * _S
    x2 = x.reshape(B, K)

    Wcat = jnp.concatenate([W_loc, W_conf], axis=0)
    Wrep = jnp.repeat(Wcat.T / _S, _S, axis=0)
    Wrep = jnp.pad(Wrep, ((0, 0), (0, 3))).astype(jnp.bfloat16)

    bm = B // 2
    bk = 8192
    out = pl.pallas_call(
        _mm_kernel,
        grid=(B // bm, K // bk),
        in_specs=[
            pl.BlockSpec((bm, bk), lambda m, k: (m, k)),
            pl.BlockSpec((bk, 8), lambda m, k: (k, 0)),
        ],
        out_specs=pl.BlockSpec((bm, 8), lambda m, k: (m, 0)),
        out_shape=jax.ShapeDtypeStruct((B, 8), jnp.float32),
        compiler_params=pltpu.CompilerParams(
            dimension_semantics=("parallel", "arbitrary")),
    )(x2, Wrep)

    loc = out[:, :2] + b_loc[None, :]
    conf = out[:, 2:5] + b_conf[None, :]
    return loc, conf


# speedup vs baseline: 5.9256x; 5.9256x over previous
import jax
import jax.numpy as jnp
from jax.experimental import pallas as pl
from jax.experimental.pallas import tpu as pltpu

_S = 64
_ROWS_PER_B = 32


def _pool_mm_kernel(x_ref, w_ref, o_ref):
    g = o_ref.shape[0]
    blk = x_ref[...]
    blk = blk.reshape(g, _ROWS_PER_B, 2, blk.shape[2])
    pooled = jnp.sum(blk, axis=(1, 2))
    o_ref[...] = jax.lax.dot_general(
        pooled.astype(jnp.bfloat16), w_ref[...],
        (((1,), (0,)), ((), ())),
        preferred_element_type=jnp.float32)


def kernel(x, start_boundaries, W_loc, b_loc, W_conf, b_conf):
    B, C = x.shape[0], x.shape[1]
    xt = x.transpose(0, 2, 3, 4, 1).reshape(B * _ROWS_PER_B, 2, C)

    Wcat = jnp.concatenate([W_loc, W_conf], axis=0)
    Wm = jnp.pad(Wcat.T / _S, ((0, 0), (0, 3))).astype(jnp.bfloat16)

    g = 8
    rb = g * _ROWS_PER_B
    out = pl.pallas_call(
        _pool_mm_kernel,
        grid=(B // g,),
        in_specs=[
            pl.BlockSpec((rb, 2, C), lambda i: (i, 0, 0)),
            pl.BlockSpec((C, 8), lambda i: (0, 0)),
        ],
        out_specs=pl.BlockSpec((g, 8), lambda i: (i, 0)),
        out_shape=jax.ShapeDtypeStruct((B, 8), jnp.float32),
        compiler_params=pltpu.CompilerParams(
            dimension_semantics=("parallel",)),
    )(xt, Wm)

    loc = out[:, :2] + b_loc[None, :]
    conf = out[:, 2:5] + b_conf[None, :]
    return loc, conf


# speedup vs baseline: 9.8554x; 1.6632x over previous
import jax
import jax.numpy as jnp
from jax.experimental import pallas as pl
from jax.experimental.pallas import tpu as pltpu

_S = 64
_ROWS_PER_B = 32
_NGRP = 16


def _pool_mm_kernel(x_ref, w_ref, o_ref):
    g = o_ref.shape[0]
    blk = x_ref[...]
    blk = blk.reshape(g, _ROWS_PER_B, 2 * _NGRP, 128)
    s1 = jnp.sum(blk, axis=1)
    s2 = s1.reshape(g, _NGRP, 2, 128).sum(axis=2)
    sb = s2.astype(jnp.bfloat16)
    acc = jnp.zeros((g, 8), jnp.float32)
    for grp in range(_NGRP):
        acc = acc + jax.lax.dot_general(
            sb[:, grp, :], w_ref[grp],
            (((1,), (0,)), ((), ())),
            preferred_element_type=jnp.float32)
    o_ref[...] = acc


def kernel(x, start_boundaries, W_loc, b_loc, W_conf, b_conf):
    B, C = x.shape[0], x.shape[1]
    xt = (x.transpose(0, 2, 3, 4, 1)
          .reshape(B * _ROWS_PER_B, 2, _NGRP, 128)
          .transpose(0, 2, 1, 3)
          .reshape(B * _ROWS_PER_B, 2 * _NGRP, 128))

    Wcat = jnp.concatenate([W_loc, W_conf], axis=0)
    W3 = jnp.pad((Wcat / _S).T.reshape(_NGRP, 128, 5),
                 ((0, 0), (0, 0), (0, 3))).astype(jnp.bfloat16)

    g = 16
    rb = g * _ROWS_PER_B
    out = pl.pallas_call(
        _pool_mm_kernel,
        grid=(B // g,),
        in_specs=[
            pl.BlockSpec((rb, 2 * _NGRP, 128), lambda i: (i, 0, 0)),
            pl.BlockSpec((_NGRP, 128, 8), lambda i: (0, 0, 0)),
        ],
        out_specs=pl.BlockSpec((g, 8), lambda i: (i, 0)),
        out_shape=jax.ShapeDtypeStruct((B, 8), jnp.float32),
        compiler_params=pltpu.CompilerParams(
            dimension_semantics=("parallel",)),
    )(xt, W3)

    loc = out[:, :2] + b_loc[None, :]
    conf = out[:, 2:5] + b_conf[None, :]
    return loc, conf
